# baseline (speedup 1.0000x reference)
import functools

import jax
import jax.numpy as jnp
from jax.experimental import pallas as pl
from jax.experimental.pallas import tpu as pltpu


def _round_up(v, m):
    return ((v + m - 1) // m) * m


def _mlp_kernel(x_ref, w1_ref, b1_ref, w2_ref, b2_ref, o_ref, *, sub):
    j = pl.program_id(1)
    rows = pl.ds(j * sub, sub)
    h = jnp.dot(x_ref[rows, :], w1_ref[...],
                preferred_element_type=jnp.float32)
    h = h + b1_ref[...]
    h = h * jax.nn.sigmoid(h)
    out = jnp.dot(h, w2_ref[...], preferred_element_type=jnp.float32)
    o_ref[...] = (out + b2_ref[...]).astype(o_ref.dtype)


def kernel(w1, b1, w2, b2, x):
    B, Din = x.shape
    D = w1.shape[1]

    Dp = _round_up(D, 128)
    TM = 2048 if B % 4096 == 0 else _round_up(min(512, B), 8)
    Bp = _round_up(B, TM)

    xp = x if Bp == B else jnp.pad(x, ((0, Bp - B), (0, 0)))
    w1p = w1 if Dp == D else jnp.pad(w1, ((0, 0), (0, Dp - D)))
    w2p = w2 if Dp == D else jnp.pad(w2, ((0, Dp - D), (0, Dp - D)))
    b1p = (b1 if Dp == D else jnp.pad(b1, (0, Dp - D))).reshape(1, Dp)
    b2p = (b2 if Dp == D else jnp.pad(b2, (0, Dp - D))).reshape(1, Dp)

    sub = 512 if TM % 512 == 0 else TM
    out = pl.pallas_call(
        functools.partial(_mlp_kernel, sub=sub),
        out_shape=jax.ShapeDtypeStruct((Bp, Dp), x.dtype),
        grid=(Bp // TM, TM // sub),
        in_specs=[
            pl.BlockSpec((TM, Din), lambda i, j: (i, 0)),
            pl.BlockSpec((Din, Dp), lambda i, j: (0, 0)),
            pl.BlockSpec((1, Dp), lambda i, j: (0, 0)),
            pl.BlockSpec((Dp, Dp), lambda i, j: (0, 0)),
            pl.BlockSpec((1, Dp), lambda i, j: (0, 0)),
        ],
        out_specs=pl.BlockSpec((sub, Dp), lambda i, j: (i * (TM // sub) + j, 0)),
        compiler_params=pltpu.CompilerParams(
            dimension_semantics=("parallel", "arbitrary"),
            vmem_limit_bytes=60 * 1024 * 1024,
        ),
    )(xp, w1p, b1p, w2p, b2p)

    return out[:B, :D]


# speedup vs baseline: 1.1922x; 1.1922x over previous
import functools

import jax
import jax.numpy as jnp
from jax.experimental import pallas as pl
from jax.experimental.pallas import tpu as pltpu


def _round_up(v, m):
    return ((v + m - 1) // m) * m


def _mlp_kernel(x_ref, w1_ref, b1_ref, w2_ref, b2_ref, o_ref, vmem_pad):
    tm = x_ref.shape[0]
    sub = 512 if tm % 512 == 0 else tm
    for j in range(tm // sub):
        rows = pl.ds(j * sub, sub)
        h = jnp.dot(x_ref[rows, :], w1_ref[...],
                    preferred_element_type=jnp.float32)
        h = h + b1_ref[...]
        h = h * jax.nn.sigmoid(h)
        out = jnp.dot(h, w2_ref[...], preferred_element_type=jnp.float32)
        o_ref[rows, :] = (out + b2_ref[...]).astype(o_ref.dtype)


def kernel(w1, b1, w2, b2, x):
    B, Din = x.shape
    D = w1.shape[1]

    Dp = _round_up(D, 128)
    TM = 2048 if B % 4096 == 0 else _round_up(min(512, B), 8)
    Bp = _round_up(B, TM)

    xp = x if Bp == B else jnp.pad(x, ((0, Bp - B), (0, 0)))
    w1p = w1 if Dp == D else jnp.pad(w1, ((0, 0), (0, Dp - D)))
    w2p = w2 if Dp == D else jnp.pad(w2, ((0, Dp - D), (0, Dp - D)))
    b1p = (b1 if Dp == D else jnp.pad(b1, (0, Dp - D))).reshape(1, Dp)
    b2p = (b2 if Dp == D else jnp.pad(b2, (0, Dp - D))).reshape(1, Dp)

    out = pl.pallas_call(
        _mlp_kernel,
        out_shape=jax.ShapeDtypeStruct((Bp, Dp), x.dtype),
        grid=(Bp // TM,),
        in_specs=[
            pl.BlockSpec((TM, Din), lambda i: (i, 0)),
            pl.BlockSpec((Din, Dp), lambda i: (0, 0)),
            pl.BlockSpec((1, Dp), lambda i: (0, 0)),
            pl.BlockSpec((Dp, Dp), lambda i: (0, 0)),
            pl.BlockSpec((1, Dp), lambda i: (0, 0)),
        ],
        out_specs=pl.BlockSpec((TM, Dp), lambda i: (i, 0)),
        scratch_shapes=[
            pltpu.VMEM((3840, 1024), jnp.float32),
        ],
        compiler_params=pltpu.CompilerParams(
            dimension_semantics=("parallel",),
            vmem_limit_bytes=60 * 1024 * 1024,
        ),
    )(xp, w1p, b1p, w2p, b2p)

    return out[:B, :D]
